# Initial kernel scaffold; baseline (speedup 1.0000x reference)
import jax
import jax.numpy as jnp
from jax.experimental import pallas as pl


def kernel(edge_index, edge_type, heads, relations, tails, entity_emb, basis0, att0, self_w0, bias0, basis1, att1, self_w1, bias1, rel_emb):
    raise NotImplementedError("write your pallas kernel here")


# speedup vs baseline: 25.7492x; 25.7492x over previous
import functools

import jax
import jax.numpy as jnp
from jax import lax
from jax.experimental import pallas as pl
from jax.experimental.pallas import tpu as pltpu
from jax.experimental.pallas import tpu_sc as plsc

NC = 2
NS = 16
NW = NC * NS
LANES = 16
CH = 80

_MESH = plsc.VectorSubcoreMesh(core_axis_name="c", subcore_axis_name="s")


def _make_norm_kernel(E, NR):
    eps = E // NS
    epw = E // NW
    zps = NR // NS
    zpad = ((zps + LANES - 1) // LANES) * LANES

    @functools.partial(
        pl.kernel,
        out_type=jax.ShapeDtypeStruct((E,), jnp.float32),
        mesh=_MESH,
        scratch_types=[
            pltpu.VMEM_SHARED((NR,), jnp.float32),
            pltpu.VMEM((CH,), jnp.int32),
            pltpu.VMEM((CH,), jnp.float32),
            pltpu.VMEM((CH,), jnp.float32),
            pltpu.VMEM((CH,), jnp.float32),
            pltpu.VMEM((zpad,), jnp.float32),
        ],
    )
    def norm_kernel(idx_deg, norm_out, deg_sp, idxb, degb, onesb, normb, zbuf):
        c = lax.axis_index("c")
        s = lax.axis_index("s")
        wid = s * NC + c

        def fill_ones(i, carry):
            onesb[pl.ds(i * LANES, LANES)] = jnp.full((LANES,), 1.0, jnp.float32)
            return carry

        lax.fori_loop(0, CH // LANES, fill_ones, 0)

        def fill_zero(i, carry):
            zbuf[pl.ds(i * LANES, LANES)] = jnp.zeros((LANES,), jnp.float32)
            return carry

        lax.fori_loop(0, zpad // LANES, fill_zero, 0)
        pltpu.sync_copy(zbuf.at[pl.ds(0, zps)],
                        deg_sp.at[pl.ds(s * zps, zps)])
        plsc.subcore_barrier()

        def deg_chunk(g, carry):
            base = s * eps + g * CH
            pltpu.sync_copy(idx_deg.at[pl.ds(base, CH)], idxb)
            pltpu.sync_copy(onesb, deg_sp.at[idxb], add=True)
            return carry

        lax.fori_loop(0, eps // CH, deg_chunk, 0)
        plsc.subcore_barrier()

        def norm_chunk(g, carry):
            base = wid * epw + g * CH
            pltpu.sync_copy(idx_deg.at[pl.ds(base, CH)], idxb)
            pltpu.sync_copy(deg_sp.at[idxb], degb)

            def recip(i, inner):
                sl = pl.ds(i * LANES, LANES)
                normb[sl] = 1.0 / degb[sl]
                return inner

            lax.fori_loop(0, CH // LANES, recip, 0)
            pltpu.sync_copy(normb, norm_out.at[pl.ds(base, CH)])
            return carry

        lax.fori_loop(0, epw // CH, norm_chunk, 0)

    return norm_kernel


def _make_msg_kernel(N, E, D):
    epw = E // NW
    rps = N // NS

    @functools.partial(
        pl.kernel,
        out_type=[jax.ShapeDtypeStruct((N, D), jnp.float32),
                  jax.ShapeDtypeStruct((N, D), jnp.float32)],
        mesh=_MESH,
        scratch_types=[
            pltpu.VMEM_SHARED((N, D), jnp.float32),
            pltpu.VMEM((CH,), jnp.int32),
            pltpu.VMEM((CH,), jnp.int32),
            pltpu.VMEM((CH,), jnp.float32),
            pltpu.VMEM((CH, D), jnp.float32),
            pltpu.VMEM((LANES, D), jnp.float32),
            pltpu.SemaphoreType.DMA,
        ],
    )
    def msg_kernel(y, idx_src, dst, norm, out0, out1,
                   out_sp, idxb, dstb, normb, rows, zbuf, sem):
        c = lax.axis_index("c")
        s = lax.axis_index("s")
        wid = s * NC + c

        def fill_zero(i, carry):
            for j in range(D // LANES):
                zbuf[i, pl.ds(j * LANES, LANES)] = jnp.zeros((LANES,),
                                                             jnp.float32)
            return carry

        lax.fori_loop(0, LANES, fill_zero, 0)
        nchunk = N // LANES

        def zchunk(k, carry):
            r0 = (s + k * NS) * LANES

            @pl.when(r0 < N)
            def _():
                pltpu.sync_copy(zbuf, out_sp.at[pl.ds(r0, LANES)])

            return carry

        lax.fori_loop(0, (nchunk + NS - 1) // NS, zchunk, 0)
        plsc.subcore_barrier()

        def chunk(g, carry):
            base = wid * epw + g * CH
            pltpu.sync_copy(idx_src.at[pl.ds(base, CH)], idxb)
            pltpu.sync_copy(norm.at[pl.ds(base, CH)], normb)
            pltpu.sync_copy(dst.at[pl.ds(base, CH)], dstb)
            pltpu.async_copy(y.at[idxb], rows, sem).wait()

            def scale(g2, inner):
                e0 = g2 * LANES
                nv16 = normb[pl.ds(e0, LANES)]
                for i in range(LANES):
                    nv = nv16[i]
                    for j in range(D // LANES):
                        sl = pl.ds(j * LANES, LANES)
                        rows[e0 + i, sl] = rows[e0 + i, sl] * nv
                return inner

            lax.fori_loop(0, CH // LANES, scale, 0)
            pltpu.sync_copy(rows, out_sp.at[dstb], add=True)
            return carry

        lax.fori_loop(0, epw // CH, chunk, 0)
        plsc.subcore_barrier()

        def out_chunk(k, carry):
            r0 = (s + k * NS) * LANES

            @pl.when(r0 < N)
            def _():
                @pl.when(c == 0)
                def _():
                    pltpu.sync_copy(out_sp.at[pl.ds(r0, LANES)],
                                    out0.at[pl.ds(r0, LANES)])

                @pl.when(c == 1)
                def _():
                    pltpu.sync_copy(out_sp.at[pl.ds(r0, LANES)],
                                    out1.at[pl.ds(r0, LANES)])

            return carry

        lax.fori_loop(0, (nchunk + NS - 1) // NS, out_chunk, 0)

    return msg_kernel


def _make_decode_kernel(N, D, Q):
    qpw = Q // NW

    @functools.partial(
        pl.kernel,
        out_type=jax.ShapeDtypeStruct((Q, LANES), jnp.float32),
        mesh=_MESH,
        scratch_types=[
            pltpu.VMEM((qpw,), jnp.int32),
            pltpu.VMEM((qpw,), jnp.int32),
            pltpu.VMEM((qpw,), jnp.int32),
            pltpu.VMEM((qpw, D), jnp.float32),
            pltpu.VMEM((qpw, D), jnp.float32),
            pltpu.VMEM((qpw, D), jnp.float32),
            pltpu.VMEM((qpw, LANES), jnp.float32),
            pltpu.SemaphoreType.DMA,
        ],
    )
    def decode_kernel(h, rel_emb, heads, rels, tails, scores,
                      hib, rib, tib, hrows, rrows, trows, outb, sem):
        c = lax.axis_index("c")
        s = lax.axis_index("s")
        wid = s * NC + c
        base = wid * qpw

        pltpu.sync_copy(heads.at[pl.ds(base, qpw)], hib)
        pltpu.sync_copy(rels.at[pl.ds(base, qpw)], rib)
        pltpu.sync_copy(tails.at[pl.ds(base, qpw)], tib)
        pltpu.async_copy(h.at[hib], hrows, sem).wait()
        pltpu.async_copy(rel_emb.at[rib], rrows, sem).wait()
        pltpu.async_copy(h.at[tib], trows, sem).wait()

        def one(q, carry):
            acc = jnp.zeros((LANES,), jnp.float32)
            for j in range(D // LANES):
                sl = pl.ds(j * LANES, LANES)
                acc = acc + (hrows[q, sl] * rrows[q, sl] * trows[q, sl])
            outb[q, :] = acc
            return carry

        lax.fori_loop(0, qpw, one, 0)
        pltpu.sync_copy(outb, scores.at[pl.ds(base, qpw)])

    return decode_kernel


def _wcat_body(att_ref, basis_ref, out_ref, *, R, NB, D):
    for r in range(R):
        acc = att_ref[r, 0] * basis_ref[0]
        for b in range(1, NB):
            acc = acc + att_ref[r, b] * basis_ref[b]
        out_ref[:, r * D:(r + 1) * D] = acc


def _wcat(att, basis):
    R, NB = att.shape
    D = basis.shape[-1]
    return pl.pallas_call(
        functools.partial(_wcat_body, R=R, NB=NB, D=D),
        out_shape=jax.ShapeDtypeStruct((D, R * D), jnp.float32),
        in_specs=[pl.BlockSpec(memory_space=pltpu.SMEM),
                  pl.BlockSpec((NB, D, D), lambda: (0, 0, 0))],
        out_specs=pl.BlockSpec((D, R * D), lambda: (0, 0)),
    )(att, basis)


def _mm_body(x_ref, w_ref, o_ref):
    o_ref[...] = jnp.dot(x_ref[...], w_ref[...],
                         preferred_element_type=jnp.float32,
                         precision=lax.Precision.HIGHEST)


def _mm(x, w, bn):
    n, d = x.shape
    m = w.shape[1]
    return pl.pallas_call(
        _mm_body,
        grid=(n // bn,),
        in_specs=[pl.BlockSpec((bn, d), lambda i: (i, 0)),
                  pl.BlockSpec((d, m), lambda i: (0, 0))],
        out_specs=pl.BlockSpec((bn, m), lambda i: (i, 0)),
        out_shape=jax.ShapeDtypeStruct((n, m), jnp.float32),
    )(x, w)


def _self_body(p0_ref, p1_ref, x_ref, w_ref, b_ref, o_ref, *, act):
    o = p0_ref[...] + p1_ref[...] + b_ref[...]
    o = o + jnp.dot(x_ref[...], w_ref[...],
                    preferred_element_type=jnp.float32,
                    precision=lax.Precision.HIGHEST)
    if act:
        o = jnp.maximum(o, 0.0)
    o_ref[...] = o


def _lane_sum_body(p_ref, o_ref):
    o_ref[...] = jnp.sum(p_ref[...], axis=-1)


def _lane_sum(prod):
    q, l = prod.shape
    return pl.pallas_call(
        _lane_sum_body,
        out_shape=jax.ShapeDtypeStruct((q,), jnp.float32),
    )(prod)


def _self_combine(p0, p1, x, w, bias, act, bn):
    n, d = x.shape
    return pl.pallas_call(
        functools.partial(_self_body, act=act),
        grid=(n // bn,),
        in_specs=[pl.BlockSpec((bn, d), lambda i: (i, 0)),
                  pl.BlockSpec((bn, d), lambda i: (i, 0)),
                  pl.BlockSpec((bn, d), lambda i: (i, 0)),
                  pl.BlockSpec((d, d), lambda i: (0, 0)),
                  pl.BlockSpec((1, d), lambda i: (0, 0))],
        out_specs=pl.BlockSpec((bn, d), lambda i: (i, 0)),
        out_shape=jax.ShapeDtypeStruct((n, d), jnp.float32),
    )(p0, p1, x, w, bias.reshape(1, d))


def kernel(edge_index, edge_type, heads, relations, tails, entity_emb,
           basis0, att0, self_w0, bias0, basis1, att1, self_w1, bias1,
           rel_emb):
    N, D = entity_emb.shape
    E = edge_type.shape[0]
    R = att0.shape[0]
    Q = heads.shape[0]
    NR = N * R
    BN = 400

    src = edge_index[0]
    dst = edge_index[1]
    idx_src = src * R + edge_type
    idx_deg = dst * R + edge_type
    norm = _make_norm_kernel(E, NR)(idx_deg)
    msg = _make_msg_kernel(N, E, D)

    x = entity_emb
    h = x
    for basis, att, self_w, bias, act in (
            (basis0, att0, self_w0, bias0, True),
            (basis1, att1, self_w1, bias1, False)):
        wcat = _wcat(att, basis)
        y = _mm(h, wcat, BN).reshape(N * R, D)
        p0, p1 = msg(y, idx_src, dst, norm)
        h = _self_combine(p0, p1, h, self_w, bias, act, BN)

    prod = _make_decode_kernel(N, D, Q)(h, rel_emb, heads, relations, tails)
    return _lane_sum(prod)
